# baseline (speedup 1.0000x reference)
import functools

import jax
import jax.numpy as jnp
from jax import lax
from jax.experimental import pallas as pl
from jax.experimental.pallas import tpu as pltpu
from jax.experimental.pallas import tpu_sc as plsc

F32 = jnp.float32
I32 = jnp.int32

_NC = 2
_NS = 16
_NW = _NC * _NS
_CH = 128
_L = 16
_D = 128
_GRID = 8


def _cdiv(a, b):
    return -(-a // b)


def _tc1_body(x_ref, w1_ref, degp_ref, ya_ref, yb_ref):
    deg = degp_ref[0, :] + degp_ref[1, :] + 1.0
    dinv = lax.rsqrt(deg)
    xw = jnp.dot(x_ref[...], w1_ref[...], preferred_element_type=F32)
    y = xw * dinv[:, None]
    ya_ref[...] = y[:, :_D]
    yb_ref[...] = y[:, _D:]


def _tc1(xp, w1, degp):
    npad = xp.shape[0]
    r = npad // _GRID
    return pl.pallas_call(
        _tc1_body,
        grid=(_GRID,),
        in_specs=[
            pl.BlockSpec((r, _D), lambda i: (i, 0)),
            pl.BlockSpec((_D, 2 * _D), lambda i: (0, 0)),
            pl.BlockSpec((_NC, r), lambda i: (0, i)),
        ],
        out_specs=[
            pl.BlockSpec((r, _D), lambda i: (i, 0)),
            pl.BlockSpec((r, _D), lambda i: (i, 0)),
        ],
        out_shape=[jax.ShapeDtypeStruct((npad, _D), F32)] * 2,
    )(xp, w1, degp)


def _tc2_body(agg1_ref, ya_ref, yb_ref, degp_ref, b1_ref, w2_ref, y2_ref):
    deg = degp_ref[0, :] + degp_ref[1, :] + 1.0
    dinv = lax.rsqrt(deg)[:, None]
    ha = agg1_ref[0, 0] + agg1_ref[0, 1] + ya_ref[...]
    hb = agg1_ref[1, 0] + agg1_ref[1, 1] + yb_ref[...]
    h = jnp.concatenate([ha, hb], axis=1) * dinv + b1_ref[...][None, :]
    h = jnp.maximum(h, 0.0)
    xw2 = jnp.dot(h, w2_ref[...], preferred_element_type=F32)
    y2_ref[...] = xw2 * dinv


def _tc2(agg1, ya, yb, degp, b1, w2):
    npad = ya.shape[0]
    r = npad // _GRID
    return pl.pallas_call(
        _tc2_body,
        grid=(_GRID,),
        in_specs=[
            pl.BlockSpec((2, _NC, r, _D), lambda i: (0, 0, i, 0)),
            pl.BlockSpec((r, _D), lambda i: (i, 0)),
            pl.BlockSpec((r, _D), lambda i: (i, 0)),
            pl.BlockSpec((_NC, r), lambda i: (0, i)),
            pl.BlockSpec((2 * _D,), lambda i: (0,)),
            pl.BlockSpec((2 * _D, _D), lambda i: (0, 0)),
        ],
        out_specs=pl.BlockSpec((r, _D), lambda i: (i, 0)),
        out_shape=jax.ShapeDtypeStruct((npad, _D), F32),
    )(agg1, ya, yb, degp, b1, w2)


def _tc3_body(agg2_ref, y2_ref, degp_ref, b2_ref, out_ref):
    deg = degp_ref[0, :] + degp_ref[1, :] + 1.0
    dinv = lax.rsqrt(deg)[:, None]
    agg = agg2_ref[0, 0] + agg2_ref[0, 1] + y2_ref[...]
    out_ref[...] = agg * dinv + b2_ref[...][None, :]


def _tc3(agg2, y2, degp, b2):
    npad = y2.shape[0]
    r = npad // _GRID
    return pl.pallas_call(
        _tc3_body,
        grid=(_GRID,),
        in_specs=[
            pl.BlockSpec((1, _NC, r, _D), lambda i: (0, 0, i, 0)),
            pl.BlockSpec((r, _D), lambda i: (i, 0)),
            pl.BlockSpec((_NC, r), lambda i: (0, i)),
            pl.BlockSpec((_D,), lambda i: (0,)),
        ],
        out_specs=pl.BlockSpec((r, _D), lambda i: (i, 0)),
        out_shape=jax.ShapeDtypeStruct((npad, _D), F32),
    )(agg2, y2, degp, b2)


def _make_deg(nchunks, npad):
    rpt = npad // _NS
    mesh = plsc.VectorSubcoreMesh(core_axis_name="c", subcore_axis_name="s")

    @functools.partial(
        pl.kernel,
        out_type=jax.ShapeDtypeStruct((_NC, npad), F32),
        mesh=mesh,
        scratch_types=[
            pltpu.VMEM((nchunks, _EC), I32),
            pltpu.VMEM((nchunks, _EC), F32),
            pltpu.VMEM((rpt,), F32),
            pltpu.VMEM_SHARED((npad,), F32),
        ],
    )
    def deg_kernel(dst_hbm, ew_hbm, out_hbm, dst_v, ew_v, zbuf, acc_sh):
        c = lax.axis_index("c")
        s = lax.axis_index("s")
        w = c * _NS + s

        @pl.loop(0, rpt // _L)
        def _zero(g):
            zbuf[pl.ds(g * _L, _L)] = jnp.zeros((_L,), F32)

        pltpu.sync_copy(zbuf, acc_sh.at[pl.ds(s * rpt, rpt)])
        pltpu.sync_copy(dst_hbm.at[w], dst_v)
        pltpu.sync_copy(ew_hbm.at[w], ew_v)
        plsc.subcore_barrier()

        @pl.loop(0, nchunks)
        def _acc(j):
            pltpu.sync_copy(ew_v.at[j], acc_sh.at[dst_v.at[j]], add=True)

        plsc.subcore_barrier()
        pltpu.sync_copy(acc_sh.at[pl.ds(s * rpt, rpt)],
                        out_hbm.at[c, pl.ds(s * rpt, rpt)])

    return deg_kernel


_EC = 112
_NCG = 6


def _make_agg(nphases, ngrp, npad):
    rpt = npad // _NS
    npair = _NCG // 2
    mesh = plsc.VectorSubcoreMesh(core_axis_name="c", subcore_axis_name="s")

    @functools.partial(
        pl.kernel,
        out_type=jax.ShapeDtypeStruct((nphases, _NC, npad, _D), F32),
        mesh=mesh,
        scratch_types=[
            pltpu.VMEM((_NCG, _EC), I32),
            pltpu.VMEM((_NCG, _EC), I32),
            pltpu.VMEM((_NCG, _EC), F32),
            pltpu.VMEM((_EC, _D), F32),
            pltpu.VMEM((_EC, _D), F32),
            pltpu.VMEM_SHARED((npad, _D), F32),
            pltpu.SemaphoreType.DMA,
            pltpu.SemaphoreType.DMA,
            pltpu.SemaphoreType.DMA,
            pltpu.SemaphoreType.DMA,
        ],
    )
    def agg_kernel(*refs):
        tables = refs[:nphases]
        src_hbm, dst_hbm, ew_hbm, out_hbm = refs[nphases:nphases + 4]
        (src_v, dst_v, ew_v, buf0, buf1, acc_sh,
         gsem0, gsem1, ssem0, ssem1) = refs[nphases + 4:]
        c = lax.axis_index("c")
        s = lax.axis_index("s")
        w = c * _NS + s
        buf = (buf0, buf1)
        gsem = (gsem0, gsem1)
        ssem = (ssem0, ssem1)

        def scale(b, j):
            @pl.loop(0, _EC // _L)
            def _scale(g):
                ew16 = ew_v[j, pl.ds(g * _L, _L)]
                for q in range(_L):
                    svec = jnp.broadcast_to(ew16[q], (_L,))
                    i = g * _L + q
                    for k in range(_D // _L):
                        sl = pl.ds(k * _L, _L)
                        buf[b][i, sl] = buf[b][i, sl] * svec

        for p in range(nphases):
            table = tables[p]

            def gather_start(b, j):
                pltpu.async_copy(
                    table.at[src_v.at[j]], buf[b], gsem[b])

            def gather_wait(b):
                pltpu.make_async_copy(
                    table.at[src_v.at[0]], buf[b], gsem[b]).wait()

            def scatter_start(b, j):
                pltpu.async_copy(
                    buf[b], acc_sh.at[dst_v.at[j]], ssem[b], add=True,
                    priority=1)

            def scatter_drain(b):
                pltpu.make_async_copy(
                    buf[b], acc_sh.at[dst_v.at[0]], ssem[b]).wait()

            @pl.loop(0, _EC)
            def _zero(i):
                for k in range(_D // _L):
                    buf0[i, pl.ds(k * _L, _L)] = jnp.zeros((_L,), F32)

            nfull = rpt // _EC
            for m in range(nfull):
                pltpu.sync_copy(
                    buf0, acc_sh.at[pl.ds(s * rpt + m * _EC, _EC)])
            rem = rpt - nfull * _EC
            if rem:
                pltpu.sync_copy(
                    buf0.at[pl.ds(0, rem)],
                    acc_sh.at[pl.ds(s * rpt + nfull * _EC, rem)])
            plsc.subcore_barrier()

            @pl.loop(0, ngrp)
            def _grp(g0):
                pltpu.sync_copy(src_hbm.at[w, g0], src_v)
                pltpu.sync_copy(dst_hbm.at[w, g0], dst_v)
                pltpu.sync_copy(ew_hbm.at[w, g0], ew_v)
                gather_start(0, 0)

                @pl.loop(0, npair)
                def _pair(t):
                    for b in range(2):
                        j = 2 * t + b
                        o = 1 - b
                        gather_wait(b)

                        @pl.when(j > 0)
                        def _():
                            scatter_drain(o)

                        @pl.when(j + 1 < _NCG)
                        def _():
                            gather_start(o, j + 1)

                        scale(b, j)
                        scatter_start(b, j)

                scatter_drain(1)

            plsc.subcore_barrier()
            pltpu.sync_copy(acc_sh.at[pl.ds(s * rpt, rpt)],
                            out_hbm.at[p, c, pl.ds(s * rpt, rpt)])
            if p + 1 < nphases:
                plsc.subcore_barrier()

    return agg_kernel


def kernel(x, edge_index, edge_weight, W1, b1, W2, b2):
    n, din = x.shape
    hid = W1.shape[1]
    dout = W2.shape[1]
    e = edge_weight.shape[0]
    assert din == _D and hid == 2 * _D and dout == _D

    npad = _cdiv(n, 2 * _NS * _CH) * (2 * _NS * _CH)
    egrain = _NW * _NCG * _EC
    e_pad = _cdiv(e, egrain) * egrain
    ngrp = e_pad // egrain
    nchunks = ngrp * _NCG
    pad = e_pad - e

    src = edge_index[0].astype(I32)
    dst = edge_index[1].astype(I32)
    ew = edge_weight.astype(F32)
    dst, src, ew = jax.lax.sort([dst, src, ew], num_keys=1)
    pad_idx = jnp.arange(pad, dtype=I32) % n
    srcp = jnp.concatenate([src, pad_idx])
    dstp = jnp.concatenate([dst, pad_idx])
    ewp = jnp.concatenate([ew, jnp.zeros((pad,), F32)])
    src4 = srcp.reshape(_NW, ngrp, _NCG, _EC)
    dst4 = dstp.reshape(_NW, ngrp, _NCG, _EC)
    ew4 = ewp.reshape(_NW, ngrp, _NCG, _EC)
    xp = jnp.pad(x.astype(F32), ((0, npad - n), (0, 0)))

    degp = _make_deg(nchunks, npad)(
        dstp.reshape(_NW, nchunks, _EC), ewp.reshape(_NW, nchunks, _EC))
    ya, yb = _tc1(xp, W1.astype(F32), degp)
    agg1 = _make_agg(2, ngrp, npad)(ya, yb, src4, dst4, ew4)
    y2 = _tc2(agg1, ya, yb, degp, b1.astype(F32), W2.astype(F32))
    agg2 = _make_agg(1, ngrp, npad)(y2, src4, dst4, ew4)
    out = _tc3(agg2, y2, degp, b2.astype(F32))
    return out[:n]


# speedup vs baseline: 1.8007x; 1.8007x over previous
import functools

import jax
import jax.numpy as jnp
from jax import lax
from jax.experimental import pallas as pl
from jax.experimental.pallas import tpu as pltpu
from jax.experimental.pallas import tpu_sc as plsc

F32 = jnp.float32
I32 = jnp.int32

_NC = 2
_NS = 16
_NW = _NC * _NS
_CH = 128
_L = 16
_D = 128
_GRID = 8


def _cdiv(a, b):
    return -(-a // b)


def _tc1_body(x_ref, w1_ref, degp_ref, ya_ref, yb_ref):
    deg = degp_ref[0, :] + degp_ref[1, :] + 1.0
    dinv = lax.rsqrt(deg)
    xw = jnp.dot(x_ref[...], w1_ref[...], preferred_element_type=F32)
    y = xw * dinv[:, None]
    ya_ref[...] = y[:, :_D]
    yb_ref[...] = y[:, _D:]


def _tc1(xp, w1, degp):
    npad = xp.shape[0]
    r = npad // _GRID
    return pl.pallas_call(
        _tc1_body,
        grid=(_GRID,),
        in_specs=[
            pl.BlockSpec((r, _D), lambda i: (i, 0)),
            pl.BlockSpec((_D, 2 * _D), lambda i: (0, 0)),
            pl.BlockSpec((_NC, r), lambda i: (0, i)),
        ],
        out_specs=[
            pl.BlockSpec((r, _D), lambda i: (i, 0)),
            pl.BlockSpec((r, _D), lambda i: (i, 0)),
        ],
        out_shape=[jax.ShapeDtypeStruct((npad, _D), F32)] * 2,
    )(xp, w1, degp)


def _tc2_body(agg1_ref, ya_ref, yb_ref, degp_ref, b1_ref, w2_ref, y2_ref):
    deg = degp_ref[0, :] + degp_ref[1, :] + 1.0
    dinv = lax.rsqrt(deg)[:, None]
    ha = agg1_ref[0, 0] + agg1_ref[0, 1] + ya_ref[...]
    hb = agg1_ref[1, 0] + agg1_ref[1, 1] + yb_ref[...]
    h = jnp.concatenate([ha, hb], axis=1) * dinv + b1_ref[...][None, :]
    h = jnp.maximum(h, 0.0)
    xw2 = jnp.dot(h, w2_ref[...], preferred_element_type=F32)
    y2_ref[...] = xw2 * dinv


def _tc2(agg1, ya, yb, degp, b1, w2):
    npad = ya.shape[0]
    r = npad // _GRID
    return pl.pallas_call(
        _tc2_body,
        grid=(_GRID,),
        in_specs=[
            pl.BlockSpec((2, _NC, r, _D), lambda i: (0, 0, i, 0)),
            pl.BlockSpec((r, _D), lambda i: (i, 0)),
            pl.BlockSpec((r, _D), lambda i: (i, 0)),
            pl.BlockSpec((_NC, r), lambda i: (0, i)),
            pl.BlockSpec((2 * _D,), lambda i: (0,)),
            pl.BlockSpec((2 * _D, _D), lambda i: (0, 0)),
        ],
        out_specs=pl.BlockSpec((r, _D), lambda i: (i, 0)),
        out_shape=jax.ShapeDtypeStruct((npad, _D), F32),
    )(agg1, ya, yb, degp, b1, w2)


def _tc3_body(agg2_ref, y2_ref, degp_ref, b2_ref, out_ref):
    deg = degp_ref[0, :] + degp_ref[1, :] + 1.0
    dinv = lax.rsqrt(deg)[:, None]
    agg = agg2_ref[0, 0] + agg2_ref[0, 1] + y2_ref[...]
    out_ref[...] = agg * dinv + b2_ref[...][None, :]


def _tc3(agg2, y2, degp, b2):
    npad = y2.shape[0]
    r = npad // _GRID
    return pl.pallas_call(
        _tc3_body,
        grid=(_GRID,),
        in_specs=[
            pl.BlockSpec((1, _NC, r, _D), lambda i: (0, 0, i, 0)),
            pl.BlockSpec((r, _D), lambda i: (i, 0)),
            pl.BlockSpec((_NC, r), lambda i: (0, i)),
            pl.BlockSpec((_D,), lambda i: (0,)),
        ],
        out_specs=pl.BlockSpec((r, _D), lambda i: (i, 0)),
        out_shape=jax.ShapeDtypeStruct((npad, _D), F32),
    )(agg2, y2, degp, b2)


def _make_deg(nchunks, npad):
    rpt = npad // _NS
    mesh = plsc.VectorSubcoreMesh(core_axis_name="c", subcore_axis_name="s")

    @functools.partial(
        pl.kernel,
        out_type=jax.ShapeDtypeStruct((_NC, npad), F32),
        mesh=mesh,
        scratch_types=[
            pltpu.VMEM((nchunks, _EC), I32),
            pltpu.VMEM((nchunks, _EC), F32),
            pltpu.VMEM((rpt,), F32),
            pltpu.VMEM_SHARED((npad,), F32),
        ],
    )
    def deg_kernel(dst_hbm, ew_hbm, out_hbm, dst_v, ew_v, zbuf, acc_sh):
        c = lax.axis_index("c")
        s = lax.axis_index("s")
        w = c * _NS + s

        @pl.loop(0, rpt // _L)
        def _zero(g):
            zbuf[pl.ds(g * _L, _L)] = jnp.zeros((_L,), F32)

        pltpu.sync_copy(zbuf, acc_sh.at[pl.ds(s * rpt, rpt)])
        pltpu.sync_copy(dst_hbm.at[w], dst_v)
        pltpu.sync_copy(ew_hbm.at[w], ew_v)
        plsc.subcore_barrier()

        @pl.loop(0, nchunks)
        def _acc(j):
            pltpu.sync_copy(ew_v.at[j], acc_sh.at[dst_v.at[j]], add=True)

        plsc.subcore_barrier()
        pltpu.sync_copy(acc_sh.at[pl.ds(s * rpt, rpt)],
                        out_hbm.at[c, pl.ds(s * rpt, rpt)])

    return deg_kernel


_EC = 112
_NCG = 6


def _make_agg(nphases, ngrp, npad):
    rpt = npad // _NS
    npair = _NCG // 2
    mesh = plsc.VectorSubcoreMesh(core_axis_name="c", subcore_axis_name="s")

    @functools.partial(
        pl.kernel,
        out_type=jax.ShapeDtypeStruct((nphases, _NC, npad, _D), F32),
        mesh=mesh,
        scratch_types=[
            pltpu.VMEM((_NCG, _EC), I32),
            pltpu.VMEM((_NCG, _EC), I32),
            pltpu.VMEM((_NCG, _EC), F32),
            pltpu.VMEM((_EC, _D), F32),
            pltpu.VMEM((_EC, _D), F32),
            pltpu.VMEM_SHARED((npad, _D), F32),
            pltpu.SemaphoreType.DMA,
            pltpu.SemaphoreType.DMA,
            pltpu.SemaphoreType.DMA,
            pltpu.SemaphoreType.DMA,
        ],
    )
    def agg_kernel(*refs):
        tables = refs[:nphases]
        src_hbm, dst_hbm, ew_hbm, out_hbm = refs[nphases:nphases + 4]
        (src_v, dst_v, ew_v, buf0, buf1, acc_sh,
         gsem0, gsem1, ssem0, ssem1) = refs[nphases + 4:]
        c = lax.axis_index("c")
        s = lax.axis_index("s")
        w = c * _NS + s
        buf = (buf0, buf1)
        gsem = (gsem0, gsem1)
        ssem = (ssem0, ssem1)

        def scale(b, j):
            @pl.loop(0, _EC // _L)
            def _scale(g):
                ew16 = ew_v[j, pl.ds(g * _L, _L)]
                for q in range(_L):
                    svec = jnp.broadcast_to(ew16[q], (_L,))
                    i = g * _L + q
                    for k in range(_D // _L):
                        sl = pl.ds(k * _L, _L)
                        buf[b][i, sl] = buf[b][i, sl] * svec

        for p in range(nphases):
            table = tables[p]

            def gather_start(b, j):
                pltpu.async_copy(
                    table.at[src_v.at[j]], buf[b], gsem[b])

            def gather_wait(b):
                pltpu.make_async_copy(
                    table.at[src_v.at[0]], buf[b], gsem[b]).wait()

            def scatter_start(b, j):
                pltpu.async_copy(
                    buf[b], acc_sh.at[dst_v.at[j]], ssem[b], add=True,
                    priority=1)

            def scatter_drain(b):
                pltpu.make_async_copy(
                    buf[b], acc_sh.at[dst_v.at[0]], ssem[b]).wait()

            @pl.loop(0, _EC)
            def _zero(i):
                for k in range(_D // _L):
                    buf0[i, pl.ds(k * _L, _L)] = jnp.zeros((_L,), F32)

            nfull = rpt // _EC
            for m in range(nfull):
                pltpu.sync_copy(
                    buf0, acc_sh.at[pl.ds(s * rpt + m * _EC, _EC)])
            rem = rpt - nfull * _EC
            if rem:
                pltpu.sync_copy(
                    buf0.at[pl.ds(0, rem)],
                    acc_sh.at[pl.ds(s * rpt + nfull * _EC, rem)])
            plsc.subcore_barrier()

            @pl.loop(0, ngrp)
            def _grp(g0):
                pltpu.sync_copy(src_hbm.at[w, g0], src_v)
                pltpu.sync_copy(dst_hbm.at[w, g0], dst_v)
                pltpu.sync_copy(ew_hbm.at[w, g0], ew_v)
                gather_start(0, 0)

                @pl.loop(0, npair)
                def _pair(t):
                    for b in range(2):
                        j = 2 * t + b
                        o = 1 - b
                        gather_wait(b)

                        @pl.when(j > 0)
                        def _():
                            scatter_drain(o)

                        @pl.when(j + 1 < _NCG)
                        def _():
                            gather_start(o, j + 1)

                        scale(b, j)
                        scatter_start(b, j)

                scatter_drain(1)

            plsc.subcore_barrier()
            pltpu.sync_copy(acc_sh.at[pl.ds(s * rpt, rpt)],
                            out_hbm.at[p, c, pl.ds(s * rpt, rpt)])
            if p + 1 < nphases:
                plsc.subcore_barrier()

    return agg_kernel


def kernel(x, edge_index, edge_weight, W1, b1, W2, b2):
    n, din = x.shape
    hid = W1.shape[1]
    dout = W2.shape[1]
    e = edge_weight.shape[0]
    assert din == _D and hid == 2 * _D and dout == _D

    npad = _cdiv(n, 2 * _NS * _CH) * (2 * _NS * _CH)
    egrain = _NW * _NCG * _EC
    e_pad = _cdiv(e, egrain) * egrain
    ngrp = e_pad // egrain
    nchunks = ngrp * _NCG
    pad = e_pad - e

    src = edge_index[0].astype(I32)
    dst = edge_index[1].astype(I32)
    ew = edge_weight.astype(F32)
    pad_idx = jnp.arange(pad, dtype=I32) % n
    srcp = jnp.concatenate([src, pad_idx])
    dstp = jnp.concatenate([dst, pad_idx])
    ewp = jnp.concatenate([ew, jnp.zeros((pad,), F32)])
    src4 = srcp.reshape(_NW, ngrp, _NCG, _EC)
    dst4 = dstp.reshape(_NW, ngrp, _NCG, _EC)
    ew4 = ewp.reshape(_NW, ngrp, _NCG, _EC)
    xp = jnp.pad(x.astype(F32), ((0, npad - n), (0, 0)))

    degp = _make_deg(nchunks, npad)(
        dstp.reshape(_NW, nchunks, _EC), ewp.reshape(_NW, nchunks, _EC))
    ya, yb = _tc1(xp, W1.astype(F32), degp)
    agg1 = _make_agg(2, ngrp, npad)(ya, yb, src4, dst4, ew4)
    y2 = _tc2(agg1, ya, yb, degp, b1.astype(F32), W2.astype(F32))
    agg2 = _make_agg(1, ngrp, npad)(y2, src4, dst4, ew4)
    out = _tc3(agg2, y2, degp, b2.astype(F32))
    return out[:n]


# speedup vs baseline: 1.8615x; 1.0337x over previous
import functools

import jax
import jax.numpy as jnp
from jax import lax
from jax.experimental import pallas as pl
from jax.experimental.pallas import tpu as pltpu
from jax.experimental.pallas import tpu_sc as plsc

F32 = jnp.float32
I32 = jnp.int32

_NC = 2
_NS = 16
_NW = _NC * _NS
_CH = 128
_L = 16
_D = 128
_GRID = 8


def _cdiv(a, b):
    return -(-a // b)


def _tc1_body(x_ref, w1_ref, degp_ref, ya_ref, yb_ref):
    deg = degp_ref[0, :] + degp_ref[1, :] + 1.0
    dinv = lax.rsqrt(deg)
    xw = jnp.dot(x_ref[...], w1_ref[...], preferred_element_type=F32)
    y = xw * dinv[:, None]
    ya_ref[...] = y[:, :_D]
    yb_ref[...] = y[:, _D:]


def _tc1(xp, w1, degp):
    npad = xp.shape[0]
    r = npad // _GRID
    return pl.pallas_call(
        _tc1_body,
        grid=(_GRID,),
        in_specs=[
            pl.BlockSpec((r, _D), lambda i: (i, 0)),
            pl.BlockSpec((_D, 2 * _D), lambda i: (0, 0)),
            pl.BlockSpec((_NC, r), lambda i: (0, i)),
        ],
        out_specs=[
            pl.BlockSpec((r, _D), lambda i: (i, 0)),
            pl.BlockSpec((r, _D), lambda i: (i, 0)),
        ],
        out_shape=[jax.ShapeDtypeStruct((npad, _D), F32)] * 2,
    )(xp, w1, degp)


def _tc2_body(agg1_ref, ya_ref, yb_ref, degp_ref, b1_ref, w2_ref, y2_ref):
    deg = degp_ref[0, :] + degp_ref[1, :] + 1.0
    dinv = lax.rsqrt(deg)[:, None]
    ha = agg1_ref[0, 0] + agg1_ref[0, 1] + ya_ref[...]
    hb = agg1_ref[1, 0] + agg1_ref[1, 1] + yb_ref[...]
    h = jnp.concatenate([ha, hb], axis=1) * dinv + b1_ref[...][None, :]
    h = jnp.maximum(h, 0.0)
    xw2 = jnp.dot(h, w2_ref[...], preferred_element_type=F32)
    y2_ref[...] = xw2 * dinv


def _tc2(agg1, ya, yb, degp, b1, w2):
    npad = ya.shape[0]
    r = npad // _GRID
    return pl.pallas_call(
        _tc2_body,
        grid=(_GRID,),
        in_specs=[
            pl.BlockSpec((2, _NC, r, _D), lambda i: (0, 0, i, 0)),
            pl.BlockSpec((r, _D), lambda i: (i, 0)),
            pl.BlockSpec((r, _D), lambda i: (i, 0)),
            pl.BlockSpec((_NC, r), lambda i: (0, i)),
            pl.BlockSpec((2 * _D,), lambda i: (0,)),
            pl.BlockSpec((2 * _D, _D), lambda i: (0, 0)),
        ],
        out_specs=pl.BlockSpec((r, _D), lambda i: (i, 0)),
        out_shape=jax.ShapeDtypeStruct((npad, _D), F32),
    )(agg1, ya, yb, degp, b1, w2)


def _tc3_body(agg2_ref, y2_ref, degp_ref, b2_ref, out_ref):
    deg = degp_ref[0, :] + degp_ref[1, :] + 1.0
    dinv = lax.rsqrt(deg)[:, None]
    agg = agg2_ref[0, 0] + agg2_ref[0, 1] + y2_ref[...]
    out_ref[...] = agg * dinv + b2_ref[...][None, :]


def _tc3(agg2, y2, degp, b2):
    npad = y2.shape[0]
    r = npad // _GRID
    return pl.pallas_call(
        _tc3_body,
        grid=(_GRID,),
        in_specs=[
            pl.BlockSpec((1, _NC, r, _D), lambda i: (0, 0, i, 0)),
            pl.BlockSpec((r, _D), lambda i: (i, 0)),
            pl.BlockSpec((_NC, r), lambda i: (0, i)),
            pl.BlockSpec((_D,), lambda i: (0,)),
        ],
        out_specs=pl.BlockSpec((r, _D), lambda i: (i, 0)),
        out_shape=jax.ShapeDtypeStruct((npad, _D), F32),
    )(agg2, y2, degp, b2)


def _make_deg(nchunks, npad):
    rpt = npad // _NS
    mesh = plsc.VectorSubcoreMesh(core_axis_name="c", subcore_axis_name="s")

    @functools.partial(
        pl.kernel,
        out_type=jax.ShapeDtypeStruct((_NC, npad), F32),
        mesh=mesh,
        scratch_types=[
            pltpu.VMEM((nchunks, _EC), I32),
            pltpu.VMEM((nchunks, _EC), F32),
            pltpu.VMEM((rpt,), F32),
            pltpu.VMEM_SHARED((npad,), F32),
        ],
    )
    def deg_kernel(dst_hbm, ew_hbm, out_hbm, dst_v, ew_v, zbuf, acc_sh):
        c = lax.axis_index("c")
        s = lax.axis_index("s")
        w = c * _NS + s

        @pl.loop(0, rpt // _L)
        def _zero(g):
            zbuf[pl.ds(g * _L, _L)] = jnp.zeros((_L,), F32)

        pltpu.sync_copy(zbuf, acc_sh.at[pl.ds(s * rpt, rpt)])
        pltpu.sync_copy(dst_hbm.at[w], dst_v)
        pltpu.sync_copy(ew_hbm.at[w], ew_v)
        plsc.subcore_barrier()

        @pl.loop(0, nchunks)
        def _acc(j):
            pltpu.sync_copy(ew_v.at[j], acc_sh.at[dst_v.at[j]], add=True)

        plsc.subcore_barrier()
        pltpu.sync_copy(acc_sh.at[pl.ds(s * rpt, rpt)],
                        out_hbm.at[c, pl.ds(s * rpt, rpt)])

    return deg_kernel


_EC = 112
_NCG = 6


def _make_agg(nphases, ngrp, npad):
    rpt = npad // _NS
    npair = _NCG // 2
    mesh = plsc.VectorSubcoreMesh(core_axis_name="c", subcore_axis_name="s")

    @functools.partial(
        pl.kernel,
        out_type=jax.ShapeDtypeStruct((nphases, _NC, npad, _D), F32),
        mesh=mesh,
        scratch_types=[
            pltpu.VMEM((2, _NCG, _EC), I32),
            pltpu.VMEM((_NCG, _EC), F32),
            pltpu.VMEM((_EC, _D), F32),
            pltpu.VMEM((_EC, _D), F32),
            pltpu.VMEM_SHARED((npad, _D), F32),
            pltpu.SemaphoreType.DMA,
            pltpu.SemaphoreType.DMA,
            pltpu.SemaphoreType.DMA,
            pltpu.SemaphoreType.DMA,
        ],
    )
    def agg_kernel(*refs):
        tables = refs[:nphases]
        idx_hbm, ew_hbm, out_hbm = refs[nphases:nphases + 3]
        (idx_v, ew_v, buf0, buf1, acc_sh,
         gsem0, gsem1, ssem0, ssem1) = refs[nphases + 3:]
        c = lax.axis_index("c")
        s = lax.axis_index("s")
        w = c * _NS + s
        buf = (buf0, buf1)
        gsem = (gsem0, gsem1)
        ssem = (ssem0, ssem1)

        def scale(b, j):
            @pl.loop(0, _EC // _L)
            def _scale(g):
                ew16 = ew_v[j, pl.ds(g * _L, _L)]
                for q in range(_L):
                    svec = jnp.broadcast_to(ew16[q], (_L,))
                    i = g * _L + q
                    for k in range(_D // _L):
                        sl = pl.ds(k * _L, _L)
                        buf[b][i, sl] = buf[b][i, sl] * svec

        for p in range(nphases):
            table = tables[p]

            def gather_start(b, j):
                pltpu.async_copy(
                    table.at[idx_v.at[0, j]], buf[b], gsem[b])

            def gather_wait(b):
                pltpu.make_async_copy(
                    table.at[idx_v.at[0, 0]], buf[b], gsem[b]).wait()

            def scatter_start(b, j):
                pltpu.async_copy(
                    buf[b], acc_sh.at[idx_v.at[1, j]], ssem[b], add=True,
                    priority=1)

            def scatter_drain(b):
                pltpu.make_async_copy(
                    buf[b], acc_sh.at[idx_v.at[1, 0]], ssem[b]).wait()

            @pl.loop(0, _EC)
            def _zero(i):
                for k in range(_D // _L):
                    buf0[i, pl.ds(k * _L, _L)] = jnp.zeros((_L,), F32)

            nfull = rpt // _EC
            for m in range(nfull):
                pltpu.sync_copy(
                    buf0, acc_sh.at[pl.ds(s * rpt + m * _EC, _EC)])
            rem = rpt - nfull * _EC
            if rem:
                pltpu.sync_copy(
                    buf0.at[pl.ds(0, rem)],
                    acc_sh.at[pl.ds(s * rpt + nfull * _EC, rem)])
            plsc.subcore_barrier()

            @pl.loop(0, ngrp)
            def _grp(g0):
                pltpu.sync_copy(idx_hbm.at[w, g0], idx_v)
                pltpu.sync_copy(ew_hbm.at[w, g0], ew_v)
                gather_start(0, 0)

                @pl.loop(0, npair)
                def _pair(t):
                    for b in range(2):
                        j = 2 * t + b
                        o = 1 - b
                        gather_wait(b)

                        @pl.when(j > 0)
                        def _():
                            scatter_drain(o)

                        @pl.when(j + 1 < _NCG)
                        def _():
                            gather_start(o, j + 1)

                        scale(b, j)
                        scatter_start(b, j)

                scatter_drain(1)

            plsc.subcore_barrier()
            pltpu.sync_copy(acc_sh.at[pl.ds(s * rpt, rpt)],
                            out_hbm.at[p, c, pl.ds(s * rpt, rpt)])
            if p + 1 < nphases:
                plsc.subcore_barrier()

    return agg_kernel


def kernel(x, edge_index, edge_weight, W1, b1, W2, b2):
    n, din = x.shape
    hid = W1.shape[1]
    dout = W2.shape[1]
    e = edge_weight.shape[0]
    assert din == _D and hid == 2 * _D and dout == _D

    npad = _cdiv(n, 2 * _NS * _CH) * (2 * _NS * _CH)
    egrain = _NW * _NCG * _EC
    e_pad = _cdiv(e, egrain) * egrain
    ngrp = e_pad // egrain
    nchunks = ngrp * _NCG
    pad = e_pad - e

    src = edge_index[0].astype(I32)
    dst = edge_index[1].astype(I32)
    ew = edge_weight.astype(F32)
    pad_idx = jnp.arange(pad, dtype=I32) % n
    srcp = jnp.concatenate([src, pad_idx])
    dstp = jnp.concatenate([dst, pad_idx])
    ewp = jnp.concatenate([ew, jnp.zeros((pad,), F32)])
    idx3 = jnp.stack(
        [srcp.reshape(_NW, ngrp, _NCG, _EC),
         dstp.reshape(_NW, ngrp, _NCG, _EC)], axis=2)
    ew4 = ewp.reshape(_NW, ngrp, _NCG, _EC)
    xp = jnp.pad(x.astype(F32), ((0, npad - n), (0, 0)))

    degp = _make_deg(nchunks, npad)(
        dstp.reshape(_NW, nchunks, _EC), ewp.reshape(_NW, nchunks, _EC))
    ya, yb = _tc1(xp, W1.astype(F32), degp)
    agg1 = _make_agg(2, ngrp, npad)(ya, yb, idx3, ew4)
    y2 = _tc2(agg1, ya, yb, degp, b1.astype(F32), W2.astype(F32))
    agg2 = _make_agg(1, ngrp, npad)(y2, idx3, ew4)
    out = _tc3(agg2, y2, degp, b2.astype(F32))
    return out[:n]
